# Initial kernel scaffold; baseline (speedup 1.0000x reference)
import jax
import jax.numpy as jnp
from jax.experimental import pallas as pl


def kernel(position, rgb_color, opacity, quaternion_rotation, scale, rot, tran):
    raise NotImplementedError("write your pallas kernel here")


# speedup vs baseline: 3.5582x; 3.5582x over previous
import jax
import jax.numpy as jnp
from jax.experimental import pallas as pl
from jax.experimental.pallas import tpu as pltpu

_LANES = 128
_SUB = 64
_BLOCK = _SUB * _LANES


def _body(pqs_ref, rgb_ref, opa_ref, rot_ref, tran_ref,
          pimg_ref, rgb_o_ref, opa_o_ref, cov_ref):
    px = pqs_ref[0, 0]
    py = pqs_ref[1, 0]
    pz = pqs_ref[2, 0]
    qw = pqs_ref[3, 0]
    qx = pqs_ref[4, 0]
    qy = pqs_ref[5, 0]
    qz = pqs_ref[6, 0]
    sx = pqs_ref[7, 0]
    sy = pqs_ref[8, 0]
    sz = pqs_ref[9, 0]

    r = [[rot_ref[i, j] for j in range(3)] for i in range(3)]
    t0 = tran_ref[0]
    t1 = tran_ref[1]
    t2 = tran_ref[2]

    xc = px * r[0][0] + py * r[0][1] + pz * r[0][2] + t0
    yc = px * r[1][0] + py * r[1][1] + pz * r[1][2] + t1
    zc = px * r[2][0] + py * r[2][1] + pz * r[2][2] + t2

    zi = 1.0 / zc
    u = xc * zi
    v = yc * zi
    depth = jnp.sqrt(xc * xc + yc * yc + zc * zc)

    pimg_ref[0, 0] = u
    pimg_ref[1, 0] = v
    pimg_ref[2, 0] = depth

    qn = 1.0 / (jnp.sqrt(qw * qw + qx * qx + qy * qy + qz * qz) + 1e-8)
    w = qw * qn
    x = qx * qn
    y = qy * qn
    z = qz * qn
    xx = x * x
    yy = y * y
    zz = z * z
    xy = x * y
    xz = x * z
    yz = y * z
    wx = w * x
    wy = w * y
    wz = w * z
    R00 = 1.0 - 2.0 * (yy + zz)
    R01 = 2.0 * (xy - wz)
    R02 = 2.0 * (xz + wy)
    R10 = 2.0 * (xy + wz)
    R11 = 1.0 - 2.0 * (xx + zz)
    R12 = 2.0 * (yz - wx)
    R20 = 2.0 * (xz - wy)
    R21 = 2.0 * (yz + wx)
    R22 = 1.0 - 2.0 * (xx + yy)

    ax = jnp.abs(sx) + 0.0001
    ay = jnp.abs(sy) + 0.0001
    az = jnp.abs(sz) + 0.0001

    a00 = R00 * ax
    a01 = R01 * ay
    a02 = R02 * az
    a10 = R10 * ax
    a11 = R11 * ay
    a12 = R12 * az
    a20 = R20 * ax
    a21 = R21 * ay
    a22 = R22 * az
    S00 = a00 * a00 + a01 * a01 + a02 * a02
    S01 = a00 * a10 + a01 * a11 + a02 * a12
    S02 = a00 * a20 + a01 * a21 + a02 * a22
    S11 = a10 * a10 + a11 * a11 + a12 * a12
    S12 = a10 * a20 + a11 * a21 + a12 * a22
    S22 = a20 * a20 + a21 * a21 + a22 * a22

    jw00 = zi * (r[0][0] - u * r[2][0])
    jw01 = zi * (r[0][1] - u * r[2][1])
    jw02 = zi * (r[0][2] - u * r[2][2])
    jw10 = zi * (r[1][0] - v * r[2][0])
    jw11 = zi * (r[1][1] - v * r[2][1])
    jw12 = zi * (r[1][2] - v * r[2][2])

    T00 = jw00 * S00 + jw01 * S01 + jw02 * S02
    T01 = jw00 * S01 + jw01 * S11 + jw02 * S12
    T02 = jw00 * S02 + jw01 * S12 + jw02 * S22
    T10 = jw10 * S00 + jw11 * S01 + jw12 * S02
    T11 = jw10 * S01 + jw11 * S11 + jw12 * S12
    T12 = jw10 * S02 + jw11 * S12 + jw12 * S22
    c00 = T00 * jw00 + T01 * jw01 + T02 * jw02
    c01 = T00 * jw10 + T01 * jw11 + T02 * jw12
    c11 = T10 * jw10 + T11 * jw11 + T12 * jw12

    cov_ref[0, 0] = c00
    cov_ref[1, 0] = c01
    cov_ref[2, 0] = c01
    cov_ref[3, 0] = c11

    rgb_o_ref[...] = jax.nn.sigmoid(rgb_ref[...])
    opa_o_ref[...] = jax.nn.sigmoid(opa_ref[...])


def kernel(position, rgb_color, opacity, quaternion_rotation, scale, rot, tran):
    n = position.shape[0]
    g = -(-n // _BLOCK)
    mp = g * _BLOCK
    pad = mp - n

    pqs = jnp.concatenate([position, quaternion_rotation, scale], axis=1)
    pqs_t = jnp.pad(pqs, ((0, pad), (0, 0))).T.reshape(10, g, _SUB, _LANES)
    rgb_p = jnp.pad(rgb_color, ((0, pad), (0, 0))).reshape(g, 3 * _SUB, _LANES)
    opa_p = jnp.pad(opacity, ((0, pad), (0, 0))).reshape(g, _SUB, _LANES)

    out_shapes = (
        jax.ShapeDtypeStruct((3, g, _SUB, _LANES), jnp.float32),
        jax.ShapeDtypeStruct((g, 3 * _SUB, _LANES), jnp.float32),
        jax.ShapeDtypeStruct((g, _SUB, _LANES), jnp.float32),
        jax.ShapeDtypeStruct((4, g, _SUB, _LANES), jnp.float32),
    )
    grid_spec = pl.GridSpec(
        grid=(g,),
        in_specs=[
            pl.BlockSpec((10, 1, _SUB, _LANES), lambda i: (0, i, 0, 0)),
            pl.BlockSpec((1, 3 * _SUB, _LANES), lambda i: (i, 0, 0)),
            pl.BlockSpec((1, _SUB, _LANES), lambda i: (i, 0, 0)),
            pl.BlockSpec(memory_space=pltpu.SMEM),
            pl.BlockSpec(memory_space=pltpu.SMEM),
        ],
        out_specs=[
            pl.BlockSpec((3, 1, _SUB, _LANES), lambda i: (0, i, 0, 0)),
            pl.BlockSpec((1, 3 * _SUB, _LANES), lambda i: (i, 0, 0)),
            pl.BlockSpec((1, _SUB, _LANES), lambda i: (i, 0, 0)),
            pl.BlockSpec((4, 1, _SUB, _LANES), lambda i: (0, i, 0, 0)),
        ],
    )
    pimg_t, rgb_o, opa_o, cov_t = pl.pallas_call(
        _body,
        grid_spec=grid_spec,
        out_shape=out_shapes,
        compiler_params=pltpu.CompilerParams(
            dimension_semantics=("arbitrary",),
        ),
    )(pqs_t, rgb_p, opa_p, rot, tran)

    pos_img = pimg_t.reshape(3, mp).T[:n]
    rgb = rgb_o.reshape(mp, 3)[:n]
    opa = opa_o.reshape(mp, 1)[:n]
    cov_2d = cov_t.reshape(4, mp).T[:n].reshape(n, 2, 2)
    return pos_img, rgb, opa, cov_2d
